# Initial kernel scaffold; baseline (speedup 1.0000x reference)
import jax
import jax.numpy as jnp
from jax.experimental import pallas as pl


def kernel(user_id, timestamp, user_table, ts_table, buckets, mean, var):
    raise NotImplementedError("write your pallas kernel here")


# speedup vs baseline: 10.5710x; 10.5710x over previous
import jax
import jax.numpy as jnp
from jax import lax
from jax.experimental import pallas as pl
from jax.experimental.pallas import tpu as pltpu
from jax.experimental.pallas import tpu_sc as plsc

_NC, _NS, _L = 2, 16, 16
_NW = _NC * _NS
_BATCH = 16384
_BPW = _BATCH // _NW
_CHUNK = 128
_NCH = _BPW // _CHUNK
_D = 32
_NB = 1000
_OUTD = 2 * _D + 1


def _tec_body(uid_hbm, ts_hbm, utab_hbm, ttab_hbm, bkt_hbm, cst_hbm,
              out_hbm,
              uid_v, ts_v, bidx_v, urows_v, trows_v, outbuf_v, bkt_v,
              cst_v, norm_v, sem):
    wid = lax.axis_index("s") * _NC + lax.axis_index("c")
    base = wid * _BPW

    pltpu.sync_copy(uid_hbm.at[wid], uid_v)
    pltpu.sync_copy(ts_hbm.at[pl.ds(base, _BPW)], ts_v)
    pltpu.sync_copy(bkt_hbm, bkt_v)
    pltpu.sync_copy(cst_hbm, cst_v)

    mean = cst_v[0]
    inv = cst_v[1]

    for k in range(_BPW // _L):
        t = ts_v[pl.ds(k * _L, _L)]
        j = (t * jnp.float32(_NB - 1)).astype(jnp.int32)
        j = jnp.minimum(jnp.maximum(j, 0), _NB - 2)
        g0 = plsc.load_gather(bkt_v, [j])
        g1 = plsc.load_gather(bkt_v, [j + 1])
        idx = j + (t >= g0).astype(jnp.int32) + (t >= g1).astype(jnp.int32)
        c, o = divmod(k * _L, _CHUNK)
        bidx_v[c, pl.ds(o, _L)] = idx
        norm_v[pl.ds(k * _L, _L)] = (t - mean) * inv

    copies = []
    for c in range(_NCH):
        copies.append(pltpu.async_copy(
            utab_hbm.at[uid_v.at[c]],
            urows_v.at[pl.ds(c * _CHUNK, _CHUNK)], sem))
    for c in range(_NCH):
        copies.append(pltpu.async_copy(
            ttab_hbm.at[bidx_v.at[c]],
            trows_v.at[pl.ds(c * _CHUNK, _CHUNK)], sem))
    for cp in copies:
        cp.wait()

    def row_copy(r, carry):
        outbuf_v[r, pl.ds(0, _L)] = urows_v[r, pl.ds(0, _L)]
        outbuf_v[r, pl.ds(_L, _L)] = urows_v[r, pl.ds(_L, _L)]
        outbuf_v[r, pl.ds(2 * _L, _L)] = trows_v[r, pl.ds(0, _L)]
        outbuf_v[r, pl.ds(3 * _L, _L)] = trows_v[r, pl.ds(_L, _L)]
        return carry
    lax.fori_loop(0, _BPW, row_copy, 0)

    lanes = lax.iota(jnp.int32, _L)
    col = jnp.full((_L,), _OUTD - 1, jnp.int32)
    for k in range(_BPW // _L):
        plsc.store_scatter(outbuf_v, [lanes + (k * _L), col],
                           norm_v[pl.ds(k * _L, _L)])

    pltpu.sync_copy(outbuf_v, out_hbm.at[pl.ds(base, _BPW)])


def kernel(user_id, timestamp, user_table, ts_table, buckets, mean, var):
    inv = jnp.float32(1.0) / jnp.sqrt(var.astype(jnp.float32) + 1e-7)
    cst = jnp.stack([jnp.full((_L,), mean, jnp.float32),
                     jnp.full((_L,), inv, jnp.float32)])
    uid3 = user_id.reshape(_NW, _NCH, _CHUNK)

    mesh = plsc.VectorSubcoreMesh(core_axis_name="c", subcore_axis_name="s")
    f = pl.kernel(
        _tec_body,
        out_type=jax.ShapeDtypeStruct((_BATCH, _OUTD), jnp.float32),
        mesh=mesh,
        compiler_params=pltpu.CompilerParams(needs_layout_passes=False,
                                             use_tc_tiling_on_sc=False),
        scratch_types=[
            pltpu.VMEM((_NCH, _CHUNK), jnp.int32),
            pltpu.VMEM((_BPW,), jnp.float32),
            pltpu.VMEM((_NCH, _CHUNK), jnp.int32),
            pltpu.VMEM((_BPW, _D), jnp.float32),
            pltpu.VMEM((_BPW, _D), jnp.float32),
            pltpu.VMEM((_BPW, _OUTD), jnp.float32),
            pltpu.VMEM((_NB,), jnp.float32),
            pltpu.VMEM((2, _L), jnp.float32),
            pltpu.VMEM((_BPW,), jnp.float32),
            pltpu.SemaphoreType.DMA,
        ],
    )
    return f(uid3, timestamp, user_table, ts_table, buckets, cst)
